# Initial kernel scaffold; baseline (speedup 1.0000x reference)
import jax
import jax.numpy as jnp
from jax.experimental import pallas as pl


def kernel(input, edge_input, edge_list, edge_weights, W1, b1, Wk0, bk0, Wk1, bk1, W2, b2, g_in, bt_in, g_msg, bt_msg, g_upd, bt_upd, g_out, bt_out):
    raise NotImplementedError("write your pallas kernel here")


# speedup vs baseline: 2.5021x; 2.5021x over previous
import functools

import jax
import jax.numpy as jnp
import numpy as np
from jax import lax
from jax.experimental import pallas as pl
from jax.experimental.pallas import tpu as pltpu
from jax.experimental.pallas import tpu_sc as plsc

N = 10000
E = 320000
D_IN = 128
D_H = 16
D_OUT = 128
D_E = 16
KH = 32
EPS = 1e-5

NC = 2
NS = 16
NW = NC * NS
L = 16

EPT = 10240
EPAD = EPT * NW
CH = 2048
NCH = EPT // CH
RPD = 128
NPAD = 10240


def _sc_counts_body(src_hbm, out_hbm, idx_v, acc_v):
    wid = lax.axis_index("s") * NC + lax.axis_index("c")
    pltpu.sync_copy(src_hbm.at[pl.ds(pl.multiple_of(wid * EPT, EPT), EPT)],
                    idx_v)
    zeros = jnp.zeros((L,), jnp.float32)

    def zbody(i, _):
        acc_v[pl.ds(i * L, L)] = zeros
        return 0

    lax.fori_loop(0, NPAD // L, zbody, 0)
    ones = jnp.ones((L,), jnp.float32)

    def body(i, _):
        idx = idx_v[pl.ds(i * L, L)]
        plsc.addupdate_scatter(acc_v, [idx], ones)
        return 0

    lax.fori_loop(0, EPT // L, body, 0)
    pltpu.sync_copy(acc_v, out_hbm.at[wid])


def _sc_gather_body(y_hbm, src2d_hbm, g_hbm, idx_v, rows_v, sem):
    wid = lax.axis_index("s") * NC + lax.axis_index("c")

    def chunk(ci, _):
        base = pl.multiple_of(wid * EPT + ci * CH, CH)
        rbase = pl.multiple_of((wid * EPT + ci * CH) // RPD, CH // RPD)
        pltpu.sync_copy(src2d_hbm.at[pl.ds(rbase, CH // RPD)], idx_v)
        descs = [
            pltpu.async_copy(
                y_hbm.at[idx_v.at[j]],
                rows_v.at[pl.ds(j * RPD, RPD)],
                sem,
            )
            for j in range(CH // RPD)
        ]
        for d in descs:
            d.wait()
        pltpu.sync_copy(rows_v, g_hbm.at[pl.ds(base, CH)])
        return 0

    lax.fori_loop(0, NCH, chunk, 0)


def _sc_scatter_body(mw_hbm, dst2d_hbm, zero_hbm, out_hbm, idx_v, rows_v,
                     shared):
    cid = lax.axis_index("c")
    sid = lax.axis_index("s")
    wid = sid * NC + cid

    @pl.when(sid == 0)
    def _():
        pltpu.sync_copy(zero_hbm, shared)

    plsc.subcore_barrier()

    def chunk(ci, _):
        base = pl.multiple_of(wid * EPT + ci * CH, CH)
        rbase = pl.multiple_of((wid * EPT + ci * CH) // RPD, CH // RPD)
        pltpu.sync_copy(dst2d_hbm.at[pl.ds(rbase, CH // RPD)], idx_v)
        pltpu.sync_copy(mw_hbm.at[pl.ds(base, CH)], rows_v)
        for j in range(CH // RPD):
            pltpu.sync_copy(
                rows_v.at[pl.ds(j * RPD, RPD)],
                shared.at[idx_v.at[j]],
                add=True,
            )
        return 0

    lax.fori_loop(0, NCH, chunk, 0)
    plsc.subcore_barrier()

    @pl.when(sid == 0)
    def _():
        pltpu.sync_copy(shared, out_hbm.at[cid])


@functools.cache
def _sc_kernels():
    mesh = plsc.VectorSubcoreMesh(core_axis_name="c", subcore_axis_name="s",
                                  num_cores=NC, num_subcores=NS)
    sc_params = pltpu.CompilerParams(needs_layout_passes=False,
                                     use_tc_tiling_on_sc=False)
    counts = pl.kernel(
        _sc_counts_body,
        out_type=jax.ShapeDtypeStruct((NW, NPAD), jnp.float32),
        mesh=mesh,
        scratch_types=[
            pltpu.VMEM((EPT,), jnp.int32),
            pltpu.VMEM((NPAD,), jnp.float32),
        ],
        compiler_params=sc_params,
    )
    gather = pl.kernel(
        _sc_gather_body,
        out_type=jax.ShapeDtypeStruct((EPAD, D_H), jnp.float32),
        mesh=mesh,
        scratch_types=[
            pltpu.VMEM((CH // RPD, RPD), jnp.int32),
            pltpu.VMEM((CH, D_H), jnp.float32),
            pltpu.SemaphoreType.DMA,
        ],
        compiler_params=sc_params,
    )
    scatter = pl.kernel(
        _sc_scatter_body,
        out_type=jax.ShapeDtypeStruct((NC, NPAD, D_H), jnp.float32),
        mesh=mesh,
        scratch_types=[
            pltpu.VMEM((CH // RPD, RPD), jnp.int32),
            pltpu.VMEM((CH, D_H), jnp.float32),
            pltpu.VMEM_SHARED((NPAD, D_H), jnp.float32),
        ],
        compiler_params=sc_params,
    )
    return counts, gather, scatter


def _tc_pre_body(x_ref, cnt_ref, w1_ref, b1_ref, gi_ref, bi_ref, gm_ref,
                 bm_ref, y_ref):
    x = x_ref[...]
    m = jnp.mean(x, axis=0, keepdims=True)
    v = jnp.mean(x * x, axis=0, keepdims=True) - m * m
    xn = gi_ref[...] * (x - m) / jnp.sqrt(v + EPS) + bi_ref[...]
    xr = jnp.maximum(xn, 0.0)
    x1 = jnp.dot(xr, w1_ref[...], preferred_element_type=jnp.float32)
    x1 = x1 + b1_ref[...]
    cnt = jnp.sum(cnt_ref[...], axis=0, keepdims=True)
    c = cnt[:, :N]
    s1 = jnp.dot(c, x1, preferred_element_type=jnp.float32)
    s2 = jnp.dot(c, x1 * x1, preferred_element_type=jnp.float32)
    mm = s1 / E
    mv = s2 / E - mm * mm
    y = gm_ref[...] * (x1 - mm) / jnp.sqrt(mv + EPS) + bm_ref[...]
    y_ref[...] = jnp.maximum(y, 0.0)


def _tc_edge_body(ei_ref, g_ref, w_ref, wk0_ref, bk0_ref, wq_ref, w0c_ref,
                  r_ref, s_ref, bb_ref, bk1h_ref, mw_ref):
    h = jnp.maximum(
        jnp.dot(ei_ref[...], wk0_ref[...], preferred_element_type=jnp.float32)
        + bk0_ref[...], 0.0)
    msg = g_ref[...]
    q = jnp.dot(h, wq_ref[...], preferred_element_type=jnp.float32)
    mex = jnp.dot(msg, r_ref[...], preferred_element_type=jnp.float32)
    mo = jnp.dot(q * mex, s_ref[...], preferred_element_type=jnp.float32)
    mo = mo + jnp.dot(msg, bb_ref[...], preferred_element_type=jnp.float32)
    mo = mo + jnp.dot(h, w0c_ref[...], preferred_element_type=jnp.float32)
    mo = mo + bk1h_ref[...]
    mw_ref[...] = mo * w_ref[...]


def _tc_post_body(up_ref, w2_ref, b2_ref, gu_ref, bu_ref, go_ref, bo_ref,
                  o_ref):
    u = jnp.sum(up_ref[...], axis=0)[:N, :]
    m = jnp.mean(u, axis=0, keepdims=True)
    v = jnp.mean(u * u, axis=0, keepdims=True) - m * m
    un = gu_ref[...] * (u - m) / jnp.sqrt(v + EPS) + bu_ref[...]
    ur = jnp.maximum(un, 0.0)
    o = jnp.dot(ur, w2_ref[...], preferred_element_type=jnp.float32)
    o = o + b2_ref[...]
    mo = jnp.mean(o, axis=0, keepdims=True)
    vo = jnp.mean(o * o, axis=0, keepdims=True) - mo * mo
    o_ref[...] = go_ref[...] * (o - mo) / jnp.sqrt(vo + EPS) + bo_ref[...]


def _zero_map(i):
    return (0, 0)


def _edge_call(ei_p, g, w2d, wk0, bk0h, wq, w0c, rj, sj, bb, bk1h):
    grid = (EPAD // CH,)

    def tile(i):
        return (i, 0)

    return pl.pallas_call(
        _tc_edge_body,
        grid=grid,
        in_specs=[
            pl.BlockSpec((CH, D_E), tile),
            pl.BlockSpec((CH, D_H), tile),
            pl.BlockSpec((CH, 1), tile),
            pl.BlockSpec((D_E, KH), _zero_map),
            pl.BlockSpec((1, KH), _zero_map),
            pl.BlockSpec((KH, D_H * D_H), _zero_map),
            pl.BlockSpec((KH, D_H), _zero_map),
            pl.BlockSpec((D_H, D_H * D_H), _zero_map),
            pl.BlockSpec((D_H * D_H, D_H), _zero_map),
            pl.BlockSpec((D_H, D_H), _zero_map),
            pl.BlockSpec((1, D_H), _zero_map),
        ],
        out_specs=pl.BlockSpec((CH, D_H), tile),
        out_shape=jax.ShapeDtypeStruct((EPAD, D_H), jnp.float32),
        compiler_params=pltpu.CompilerParams(
            dimension_semantics=("arbitrary",)),
    )(ei_p, g, w2d, wk0, bk0h, wq, w0c, rj, sj, bb, bk1h)


_R_NP = np.zeros((D_H, D_H * D_H), np.float32)
for _k in range(D_H):
    _R_NP[_k, _k * D_H:(_k + 1) * D_H] = 1.0
_S_NP = np.zeros((D_H * D_H, D_H), np.float32)
for _c in range(D_H * D_H):
    _S_NP[_c, _c % D_H] = 1.0


def kernel(input, edge_input, edge_list, edge_weights, W1, b1, Wk0, bk0, Wk1,
           bk1, W2, b2, g_in, bt_in, g_msg, bt_msg, g_upd, bt_upd, g_out,
           bt_out):
    f32 = jnp.float32
    npad_e = EPAD - E
    src = edge_list[:, 0]
    dst = edge_list[:, 1]
    src_cnt = jnp.concatenate([src, jnp.full((npad_e,), NPAD - 1, jnp.int32)])
    src_g = jnp.concatenate([src, jnp.zeros((npad_e,), jnp.int32)])
    dst_p = jnp.concatenate([dst, jnp.full((npad_e,), NPAD - 1, jnp.int32)])
    src2d = src_g.reshape(EPAD // RPD, RPD)
    dst2d = dst_p.reshape(EPAD // RPD, RPD)
    ei_p = jnp.concatenate([edge_input, jnp.zeros((npad_e, D_E), f32)])
    w2d = jnp.concatenate([edge_weights, jnp.zeros((npad_e,), f32)])
    w2d = w2d.reshape(EPAD, 1)
    zero_init = jnp.zeros((NPAD, D_H), f32)

    a3 = Wk1[:, D_H:].reshape(KH, D_H, D_H)
    wq = a3.transpose(0, 2, 1).reshape(KH, D_H * D_H)
    w0c = Wk1[:, :D_H]
    bb = bk1[D_H:].reshape(D_H, D_H).T
    rj = jnp.asarray(_R_NP)
    sj = jnp.asarray(_S_NP)
    bk0h = bk0.reshape(1, KH)
    bk1h = bk1[:D_H].reshape(1, D_H)

    sc_counts, sc_gather, sc_scatter = _sc_kernels()

    cnt = sc_counts(src_cnt)

    y = pl.pallas_call(
        _tc_pre_body,
        out_shape=jax.ShapeDtypeStruct((N, D_H), f32),
    )(input, cnt, W1, b1.reshape(1, D_H), g_in.reshape(1, D_IN),
      bt_in.reshape(1, D_IN), g_msg.reshape(1, D_H), bt_msg.reshape(1, D_H))

    g = sc_gather(y, src2d)

    mw = _edge_call(ei_p, g, w2d, Wk0, bk0h, wq, w0c, rj, sj, bb, bk1h)

    up = sc_scatter(mw, dst2d, zero_init)

    out = pl.pallas_call(
        _tc_post_body,
        out_shape=jax.ShapeDtypeStruct((N, D_OUT), f32),
    )(up, W2, b2.reshape(1, D_OUT), g_upd.reshape(1, D_H),
      bt_upd.reshape(1, D_H), g_out.reshape(1, D_OUT),
      bt_out.reshape(1, D_OUT))
    return out


# speedup vs baseline: 2.5149x; 1.0051x over previous
import functools

import jax
import jax.numpy as jnp
import numpy as np
from jax import lax
from jax.experimental import pallas as pl
from jax.experimental.pallas import tpu as pltpu
from jax.experimental.pallas import tpu_sc as plsc

N = 10000
E = 320000
D_IN = 128
D_H = 16
D_OUT = 128
D_E = 16
KH = 32
EPS = 1e-5

NC = 2
NS = 16
NW = NC * NS
L = 16

EPT = E // NW
CH = 2000
NCH = EPT // CH
RPD = 125
NPAD = 10240


def _sc_counts_body(src_hbm, out_hbm, idx_v, acc_v):
    wid = lax.axis_index("s") * NC + lax.axis_index("c")
    pltpu.sync_copy(src_hbm.at[pl.ds(pl.multiple_of(wid * EPT, 8), EPT)],
                    idx_v)
    zeros = jnp.zeros((L,), jnp.float32)

    def zbody(i, _):
        acc_v[pl.ds(i * L, L)] = zeros
        return 0

    lax.fori_loop(0, NPAD // L, zbody, 0)
    ones = jnp.ones((L,), jnp.float32)

    def body(i, _):
        idx = idx_v[pl.ds(i * L, L)]
        plsc.addupdate_scatter(acc_v, [idx], ones)
        return 0

    lax.fori_loop(0, EPT // L, body, 0)
    pltpu.sync_copy(acc_v, out_hbm.at[wid])


def _sc_gather_body(y_hbm, src2d_hbm, g_hbm, idx_v, rows_v, sem):
    wid = lax.axis_index("s") * NC + lax.axis_index("c")

    def chunk(ci, _):
        base = pl.multiple_of(wid * EPT + ci * CH, 8)
        rbase = pl.multiple_of((wid * EPT + ci * CH) // RPD, CH // RPD)
        pltpu.sync_copy(src2d_hbm.at[pl.ds(rbase, CH // RPD)], idx_v)
        descs = [
            pltpu.async_copy(
                y_hbm.at[idx_v.at[j]],
                rows_v.at[pl.ds(j * RPD, RPD)],
                sem,
            )
            for j in range(CH // RPD)
        ]
        for d in descs:
            d.wait()
        pltpu.sync_copy(rows_v, g_hbm.at[pl.ds(base, CH)])
        return 0

    lax.fori_loop(0, NCH, chunk, 0)


def _sc_scatter_body(mw_hbm, dst2d_hbm, zero_hbm, out_hbm, idx_v, rows_v,
                     shared):
    cid = lax.axis_index("c")
    sid = lax.axis_index("s")
    wid = sid * NC + cid

    @pl.when(sid == 0)
    def _():
        pltpu.sync_copy(zero_hbm, shared)

    plsc.subcore_barrier()

    def chunk(ci, _):
        base = pl.multiple_of(wid * EPT + ci * CH, 8)
        rbase = pl.multiple_of((wid * EPT + ci * CH) // RPD, CH // RPD)
        pltpu.sync_copy(dst2d_hbm.at[pl.ds(rbase, CH // RPD)], idx_v)
        pltpu.sync_copy(mw_hbm.at[pl.ds(base, CH)], rows_v)
        for j in range(CH // RPD):
            pltpu.sync_copy(
                rows_v.at[pl.ds(j * RPD, RPD)],
                shared.at[idx_v.at[j]],
                add=True,
            )
        return 0

    lax.fori_loop(0, NCH, chunk, 0)
    plsc.subcore_barrier()

    @pl.when(sid == 0)
    def _():
        pltpu.sync_copy(shared, out_hbm.at[cid])


@functools.cache
def _sc_kernels():
    mesh = plsc.VectorSubcoreMesh(core_axis_name="c", subcore_axis_name="s",
                                  num_cores=NC, num_subcores=NS)
    sc_params = pltpu.CompilerParams(needs_layout_passes=False,
                                     use_tc_tiling_on_sc=False)
    counts = pl.kernel(
        _sc_counts_body,
        out_type=jax.ShapeDtypeStruct((NW, NPAD), jnp.float32),
        mesh=mesh,
        scratch_types=[
            pltpu.VMEM((EPT,), jnp.int32),
            pltpu.VMEM((NPAD,), jnp.float32),
        ],
        compiler_params=sc_params,
    )
    gather = pl.kernel(
        _sc_gather_body,
        out_type=jax.ShapeDtypeStruct((E, D_H), jnp.float32),
        mesh=mesh,
        scratch_types=[
            pltpu.VMEM((CH // RPD, RPD), jnp.int32),
            pltpu.VMEM((CH, D_H), jnp.float32),
            pltpu.SemaphoreType.DMA,
        ],
        compiler_params=sc_params,
    )
    scatter = pl.kernel(
        _sc_scatter_body,
        out_type=jax.ShapeDtypeStruct((NC, NPAD, D_H), jnp.float32),
        mesh=mesh,
        scratch_types=[
            pltpu.VMEM((CH // RPD, RPD), jnp.int32),
            pltpu.VMEM((CH, D_H), jnp.float32),
            pltpu.VMEM_SHARED((NPAD, D_H), jnp.float32),
        ],
        compiler_params=sc_params,
    )
    return counts, gather, scatter


def _tc_pre_body(x_ref, cnt_ref, w1_ref, b1_ref, gi_ref, bi_ref, gm_ref,
                 bm_ref, y_ref):
    x = x_ref[...]
    m = jnp.mean(x, axis=0, keepdims=True)
    v = jnp.mean(x * x, axis=0, keepdims=True) - m * m
    xn = gi_ref[...] * (x - m) / jnp.sqrt(v + EPS) + bi_ref[...]
    xr = jnp.maximum(xn, 0.0)
    x1 = jnp.dot(xr, w1_ref[...], preferred_element_type=jnp.float32)
    x1 = x1 + b1_ref[...]
    cnt = jnp.sum(cnt_ref[...], axis=0, keepdims=True)
    c = cnt[:, :N]
    s1 = jnp.dot(c, x1, preferred_element_type=jnp.float32)
    s2 = jnp.dot(c, x1 * x1, preferred_element_type=jnp.float32)
    mm = s1 / E
    mv = s2 / E - mm * mm
    y = gm_ref[...] * (x1 - mm) / jnp.sqrt(mv + EPS) + bm_ref[...]
    y_ref[...] = jnp.maximum(y, 0.0)


def _tc_edge_body(ei_ref, g_ref, w_ref, wk0_ref, bk0_ref, wq_ref, w0c_ref,
                  r_ref, s_ref, bb_ref, bk1h_ref, mw_ref):
    h = jnp.maximum(
        jnp.dot(ei_ref[...], wk0_ref[...], preferred_element_type=jnp.float32)
        + bk0_ref[...], 0.0)
    msg = g_ref[...]
    q = jnp.dot(h, wq_ref[...], preferred_element_type=jnp.float32)
    mex = jnp.dot(msg, r_ref[...], preferred_element_type=jnp.float32)
    mo = jnp.dot(q * mex, s_ref[...], preferred_element_type=jnp.float32)
    mo = mo + jnp.dot(msg, bb_ref[...], preferred_element_type=jnp.float32)
    mo = mo + jnp.dot(h, w0c_ref[...], preferred_element_type=jnp.float32)
    mo = mo + bk1h_ref[...]
    mw_ref[...] = mo * w_ref[...]


def _tc_post_body(up_ref, w2_ref, b2_ref, gu_ref, bu_ref, go_ref, bo_ref,
                  o_ref):
    u = jnp.sum(up_ref[...], axis=0)[:N, :]
    m = jnp.mean(u, axis=0, keepdims=True)
    v = jnp.mean(u * u, axis=0, keepdims=True) - m * m
    un = gu_ref[...] * (u - m) / jnp.sqrt(v + EPS) + bu_ref[...]
    ur = jnp.maximum(un, 0.0)
    o = jnp.dot(ur, w2_ref[...], preferred_element_type=jnp.float32)
    o = o + b2_ref[...]
    mo = jnp.mean(o, axis=0, keepdims=True)
    vo = jnp.mean(o * o, axis=0, keepdims=True) - mo * mo
    o_ref[...] = go_ref[...] * (o - mo) / jnp.sqrt(vo + EPS) + bo_ref[...]


def _zero_map(i):
    return (0, 0)


def _edge_call(ei_p, g, w2d, wk0, bk0h, wq, w0c, rj, sj, bb, bk1h):
    grid = (E // CH,)

    def tile(i):
        return (i, 0)

    return pl.pallas_call(
        _tc_edge_body,
        grid=grid,
        in_specs=[
            pl.BlockSpec((CH, D_E), tile),
            pl.BlockSpec((CH, D_H), tile),
            pl.BlockSpec((CH, 1), tile),
            pl.BlockSpec((D_E, KH), _zero_map),
            pl.BlockSpec((1, KH), _zero_map),
            pl.BlockSpec((KH, D_H * D_H), _zero_map),
            pl.BlockSpec((KH, D_H), _zero_map),
            pl.BlockSpec((D_H, D_H * D_H), _zero_map),
            pl.BlockSpec((D_H * D_H, D_H), _zero_map),
            pl.BlockSpec((D_H, D_H), _zero_map),
            pl.BlockSpec((1, D_H), _zero_map),
        ],
        out_specs=pl.BlockSpec((CH, D_H), tile),
        out_shape=jax.ShapeDtypeStruct((E, D_H), jnp.float32),
        compiler_params=pltpu.CompilerParams(
            dimension_semantics=("arbitrary",)),
    )(ei_p, g, w2d, wk0, bk0h, wq, w0c, rj, sj, bb, bk1h)


_R_NP = np.zeros((D_H, D_H * D_H), np.float32)
for _k in range(D_H):
    _R_NP[_k, _k * D_H:(_k + 1) * D_H] = 1.0
_S_NP = np.zeros((D_H * D_H, D_H), np.float32)
for _c in range(D_H * D_H):
    _S_NP[_c, _c % D_H] = 1.0


def kernel(input, edge_input, edge_list, edge_weights, W1, b1, Wk0, bk0, Wk1,
           bk1, W2, b2, g_in, bt_in, g_msg, bt_msg, g_upd, bt_upd, g_out,
           bt_out):
    f32 = jnp.float32
    src = edge_list[:, 0]
    dst = edge_list[:, 1]
    src2d = src.reshape(E // RPD, RPD)
    dst2d = dst.reshape(E // RPD, RPD)
    w2d = edge_weights.reshape(E, 1)
    zero_init = jnp.zeros((NPAD, D_H), f32)

    a3 = Wk1[:, D_H:].reshape(KH, D_H, D_H)
    wq = a3.transpose(0, 2, 1).reshape(KH, D_H * D_H)
    w0c = Wk1[:, :D_H]
    bb = bk1[D_H:].reshape(D_H, D_H).T
    rj = jnp.asarray(_R_NP)
    sj = jnp.asarray(_S_NP)
    bk0h = bk0.reshape(1, KH)
    bk1h = bk1[:D_H].reshape(1, D_H)

    sc_counts, sc_gather, sc_scatter = _sc_kernels()

    cnt = sc_counts(src)

    y = pl.pallas_call(
        _tc_pre_body,
        out_shape=jax.ShapeDtypeStruct((N, D_H), f32),
    )(input, cnt, W1, b1.reshape(1, D_H), g_in.reshape(1, D_IN),
      bt_in.reshape(1, D_IN), g_msg.reshape(1, D_H), bt_msg.reshape(1, D_H))

    g = sc_gather(y, src2d)

    mw = _edge_call(edge_input, g, w2d, Wk0, bk0h, wq, w0c, rj, sj, bb, bk1h)

    up = sc_scatter(mw, dst2d, zero_init)

    out = pl.pallas_call(
        _tc_post_body,
        out_shape=jax.ShapeDtypeStruct((N, D_OUT), f32),
    )(up, W2, b2.reshape(1, D_OUT), g_upd.reshape(1, D_H),
      bt_upd.reshape(1, D_H), g_out.reshape(1, D_OUT),
      bt_out.reshape(1, D_OUT))
    return out
